# baseline (speedup 1.0000x reference)
import functools

import jax
import jax.numpy as jnp
from jax import lax
from jax.experimental import pallas as pl
from jax.experimental.pallas import tpu as pltpu
from jax.experimental.pallas import tpu_sc as plsc

NUM_TOKENS = 16384
NUM_EXPERTS = 64
NUM_GROUPS = 8
GROUP_SIZE = NUM_EXPERTS // NUM_GROUPS
TOPK_GROUPS = 4
NCAND = TOPK_GROUPS * GROUP_SIZE
K = 8
SCALE = 2.5

NC = 2
NS = 16
L = 16
NW = NC * NS
TPW = NUM_TOKENS // NW
TILE = L
NT = TPW // TILE
NP = NT // 2
SSTR = L + 1


def _tec_kernel(logits_hbm, bias_hbm, w_hbm, id_hbm,
                raw_v, bias_v, sbuf_v, cbuf_v, gmap_v,
                wout0, iout0, wout1, iout1,
                insem, wsem0, isem0, wsem1, isem1):
    wid = lax.axis_index("s") * NC + lax.axis_index("c")
    base = wid * TPW
    half = TPW // 2

    pltpu.sync_copy(logits_hbm.at[pl.ds(base, half)], raw_v.at[pl.ds(0, half)])
    pltpu.async_copy(logits_hbm.at[pl.ds(base + half, half)],
                     raw_v.at[pl.ds(half, half)], insem)
    pltpu.sync_copy(bias_hbm, bias_v)

    iota = lax.iota(jnp.int32, L)
    neg_inf = jnp.full((L,), -jnp.inf, jnp.float32)
    bias_chunks = [bias_v[pl.ds(c * L, L)] for c in range(NUM_EXPERTS // L)]

    def process_tile(t, wout_v, iout_v):
        rows = t * TILE + iota

        for c in range(TILE * NUM_EXPERTS // L):
            tok = jnp.full((L,), c // 4, jnp.int32) + t * TILE
            v = plsc.load_gather(raw_v, [tok, (c % 4) * L + iota])
            v = v + bias_chunks[c % 4]
            dst = ((c % 4) * L + iota) * SSTR + (c // 4)
            plsc.store_scatter(sbuf_v, [dst], v)

        m1 = [neg_inf] * NUM_GROUPS
        m2 = [neg_inf] * NUM_GROUPS
        srow = []
        for e in range(NUM_EXPERTS):
            s = plsc.load_gather(sbuf_v, [iota + e * SSTR])
            srow.append(s)
            g = e // GROUP_SIZE
            m2[g] = jnp.maximum(m2[g], jnp.minimum(m1[g], s))
            m1[g] = jnp.maximum(m1[g], s)
        gs = [m1[g] + m2[g] for g in range(NUM_GROUPS)]

        sel = []
        for g in range(NUM_GROUPS):
            cnt = jnp.zeros((L,), jnp.int32)
            for h in range(NUM_GROUPS):
                if h == g:
                    continue
                beats = gs[h] > gs[g]
                if h < g:
                    beats = jnp.logical_or(beats, gs[h] == gs[g])
                cnt = cnt + jnp.where(beats, 1, 0)
            sel.append(cnt < TOPK_GROUPS)

        rank = jnp.zeros((L,), jnp.int32)
        gbase = []
        for g in range(NUM_GROUPS):
            gbase.append(rank * (GROUP_SIZE * L) + iota)
            plsc.store_scatter(gmap_v, [rank * L + iota],
                               jnp.full((L,), g, jnp.int32), mask=sel[g])
            rank = rank + jnp.where(sel[g], 1, 0)
        for e in range(NUM_EXPERTS):
            g = e // GROUP_SIZE
            plsc.store_scatter(cbuf_v, [gbase[g] + (e % GROUP_SIZE) * L],
                               srow[e], mask=sel[g])

        cand = [cbuf_v[pl.ds(i * L, L)] for i in range(NCAND)]
        ws = []
        bis = []
        for k in range(K):
            vals = list(cand)
            idxs = [jnp.full((L,), i, jnp.int32) for i in range(NCAND)]
            n = NCAND
            while n > 1:
                nv, ni = [], []
                for i in range(0, n, 2):
                    better = vals[i + 1] > vals[i]
                    nv.append(jnp.where(better, vals[i + 1], vals[i]))
                    ni.append(jnp.where(better, idxs[i + 1], idxs[i]))
                vals, idxs, n = nv, ni, n // 2
            bslot = idxs[0]
            for i in range(NCAND):
                cand[i] = jnp.where(bslot == i, neg_inf, cand[i])
            gm = plsc.load_gather(gmap_v, [(bslot // GROUP_SIZE) * L + iota])
            bi = gm * GROUP_SIZE + (bslot % GROUP_SIZE)
            ws.append(plsc.load_gather(raw_v, [rows, bi]))
            bis.append(bi)

        lvl = list(ws)
        while len(lvl) > 1:
            h = len(lvl) // 2
            lvl = [lvl[i] + lvl[i + h] for i in range(h)]
        inv = SCALE / lvl[0]
        for k in range(K):
            kcol = jnp.full((L,), k, jnp.int32)
            plsc.store_scatter(wout_v, [iota, kcol], ws[k] * inv)
            plsc.store_scatter(iout_v, [iota, kcol], bis[k])

    def pair_body(p, carry):
        tA = p * 2
        tB = tA + 1

        @pl.when(p == NP // 2)
        def _wait_in():
            pltpu.make_async_copy(
                logits_hbm.at[pl.ds(base + half, half)],
                raw_v.at[pl.ds(half, half)], insem).wait()

        @pl.when(p >= 1)
        def _wait0():
            pltpu.make_async_copy(wout0, w_hbm.at[pl.ds(base, TILE)],
                                  wsem0).wait()
            pltpu.make_async_copy(iout0, id_hbm.at[pl.ds(base, TILE)],
                                  isem0).wait()

        process_tile(tA, wout0, iout0)
        pltpu.async_copy(wout0, w_hbm.at[pl.ds(base + tA * TILE, TILE)], wsem0)
        pltpu.async_copy(iout0, id_hbm.at[pl.ds(base + tA * TILE, TILE)], isem0)

        @pl.when(p >= 1)
        def _wait1():
            pltpu.make_async_copy(wout1, w_hbm.at[pl.ds(base, TILE)],
                                  wsem1).wait()
            pltpu.make_async_copy(iout1, id_hbm.at[pl.ds(base, TILE)],
                                  isem1).wait()

        process_tile(tB, wout1, iout1)
        pltpu.async_copy(wout1, w_hbm.at[pl.ds(base + tB * TILE, TILE)], wsem1)
        pltpu.async_copy(iout1, id_hbm.at[pl.ds(base + tB * TILE, TILE)], isem1)
        return carry

    lax.fori_loop(0, NP, pair_body, 0)

    pltpu.make_async_copy(wout0, w_hbm.at[pl.ds(base, TILE)], wsem0).wait()
    pltpu.make_async_copy(iout0, id_hbm.at[pl.ds(base, TILE)], isem0).wait()
    pltpu.make_async_copy(wout1, w_hbm.at[pl.ds(base, TILE)], wsem1).wait()
    pltpu.make_async_copy(iout1, id_hbm.at[pl.ds(base, TILE)], isem1).wait()


@jax.jit
def kernel(router_logits, correction_bias):
    mesh = plsc.VectorSubcoreMesh(core_axis_name="c", subcore_axis_name="s")
    run = functools.partial(
        pl.kernel,
        out_type=(
            jax.ShapeDtypeStruct((NUM_TOKENS, K), jnp.float32),
            jax.ShapeDtypeStruct((NUM_TOKENS, K), jnp.int32),
        ),
        mesh=mesh,
        compiler_params=pltpu.CompilerParams(needs_layout_passes=False),
        scratch_types=[
            pltpu.VMEM((TPW, NUM_EXPERTS), jnp.float32),
            pltpu.VMEM((NUM_EXPERTS,), jnp.float32),
            pltpu.VMEM((NUM_EXPERTS * SSTR,), jnp.float32),
            pltpu.VMEM((NCAND * L,), jnp.float32),
            pltpu.VMEM((TOPK_GROUPS * L,), jnp.int32),
            pltpu.VMEM((TILE, K), jnp.float32),
            pltpu.VMEM((TILE, K), jnp.int32),
            pltpu.VMEM((TILE, K), jnp.float32),
            pltpu.VMEM((TILE, K), jnp.int32),
            pltpu.SemaphoreType.DMA,
            pltpu.SemaphoreType.DMA,
            pltpu.SemaphoreType.DMA,
            pltpu.SemaphoreType.DMA,
            pltpu.SemaphoreType.DMA,
        ],
    )(_tec_kernel)
    return run(router_logits, correction_bias)


# speedup vs baseline: 1.2423x; 1.2423x over previous
import functools

import jax
import jax.numpy as jnp
from jax import lax
from jax.experimental import pallas as pl
from jax.experimental.pallas import tpu as pltpu
from jax.experimental.pallas import tpu_sc as plsc

NUM_TOKENS = 16384
NUM_EXPERTS = 64
NUM_GROUPS = 8
GROUP_SIZE = NUM_EXPERTS // NUM_GROUPS
TOPK_GROUPS = 4
NCAND = TOPK_GROUPS * GROUP_SIZE
K = 8
SCALE = 2.5

NC = 2
NS = 16
L = 16
NW = NC * NS
TPW = NUM_TOKENS // NW
TILE = L
NT = TPW // TILE
HNT = NT // 4
SSTR = L + 1


def _tec_kernel(logits_hbm, bias_hbm, w_hbm, id_hbm,
                raw_v, bias_v, wout_v, iout_v, sbuf_v, cbuf_v, gmap_v):
    wid = lax.axis_index("s") * NC + lax.axis_index("c")
    base = wid * TPW

    pltpu.sync_copy(logits_hbm.at[pl.ds(base, TPW)], raw_v)
    pltpu.sync_copy(bias_hbm, bias_v)

    iota = lax.iota(jnp.int32, L)
    neg_inf = jnp.full((L,), -jnp.inf, jnp.float32)
    bias_chunks = [bias_v[pl.ds(c * L, L)] for c in range(NUM_EXPERTS // L)]

    def tile_body(t, carry):
        rows = t * TILE + iota
        coliota = iota

        for c in range(TILE * NUM_EXPERTS // L):
            tok = jnp.full((L,), c // 4, jnp.int32) + t * TILE
            v = plsc.load_gather(raw_v, [tok, (c % 4) * L + coliota])
            v = v + bias_chunks[c % 4]
            dst = ((c % 4) * L + iota) * SSTR + (c // 4)
            plsc.store_scatter(sbuf_v, [dst], v)

        m1 = [neg_inf] * NUM_GROUPS
        m2 = [neg_inf] * NUM_GROUPS
        for e in range(NUM_EXPERTS):
            s = plsc.load_gather(sbuf_v, [iota + e * SSTR])
            g = e // GROUP_SIZE
            m2[g] = jnp.maximum(m2[g], jnp.minimum(m1[g], s))
            m1[g] = jnp.maximum(m1[g], s)
        gs = [m1[g] + m2[g] for g in range(NUM_GROUPS)]

        sel = []
        for g in range(NUM_GROUPS):
            cnt = jnp.zeros((L,), jnp.int32)
            for h in range(NUM_GROUPS):
                if h == g:
                    continue
                beats = gs[h] > gs[g]
                if h < g:
                    beats = jnp.logical_or(beats, gs[h] == gs[g])
                cnt = cnt + jnp.where(beats, 1, 0)
            sel.append(cnt < TOPK_GROUPS)

        rank = jnp.zeros((L,), jnp.int32)
        gbase = []
        for g in range(NUM_GROUPS):
            gbase.append(rank * (GROUP_SIZE * L) + iota)
            plsc.store_scatter(gmap_v, [rank * L + iota],
                               jnp.full((L,), g, jnp.int32), mask=sel[g])
            rank = rank + jnp.where(sel[g], 1, 0)
        for e in range(NUM_EXPERTS):
            g = e // GROUP_SIZE
            plsc.store_scatter(cbuf_v, [gbase[g] + (e % GROUP_SIZE) * L],
                               plsc.load_gather(sbuf_v, [iota + e * SSTR]),
                               mask=sel[g])

        cand = [cbuf_v[pl.ds(i * L, L)] for i in range(NCAND)]
        ws = []
        bis = []
        for k in range(K):
            vals = list(cand)
            idxs = [jnp.full((L,), i, jnp.int32) for i in range(NCAND)]
            n = NCAND
            while n > 1:
                nv, ni = [], []
                for i in range(0, n, 2):
                    better = vals[i + 1] > vals[i]
                    nv.append(jnp.where(better, vals[i + 1], vals[i]))
                    ni.append(jnp.where(better, idxs[i + 1], idxs[i]))
                vals, idxs, n = nv, ni, n // 2
            bslot = idxs[0]
            for i in range(NCAND):
                cand[i] = jnp.where(bslot == i, neg_inf, cand[i])
            gm = plsc.load_gather(gmap_v, [(bslot // GROUP_SIZE) * L + iota])
            bi = gm * GROUP_SIZE + (bslot % GROUP_SIZE)
            ws.append(plsc.load_gather(raw_v, [rows, bi]))
            bis.append(bi)

        lvl = list(ws)
        while len(lvl) > 1:
            h = len(lvl) // 2
            lvl = [lvl[i] + lvl[i + h] for i in range(h)]
        wsum = lvl[0]
        inv = SCALE / wsum
        rows = (t % HNT) * TILE + iota
        for k in range(K):
            kcol = jnp.full((L,), k, jnp.int32)
            plsc.store_scatter(wout_v, [rows, kcol], ws[k] * inv)
            plsc.store_scatter(iout_v, [rows, kcol], bis[k])

        @pl.when(t % HNT == HNT - 1)
        def _flush():
            hbase = base + (t // HNT) * (TPW // 4)
            pltpu.sync_copy(wout_v, w_hbm.at[pl.ds(hbase, TPW // 4)])
            pltpu.sync_copy(iout_v, id_hbm.at[pl.ds(hbase, TPW // 4)])
        return carry

    lax.fori_loop(0, NT, tile_body, 0)


@jax.jit
def kernel(router_logits, correction_bias):
    mesh = plsc.VectorSubcoreMesh(core_axis_name="c", subcore_axis_name="s")
    run = functools.partial(
        pl.kernel,
        out_type=(
            jax.ShapeDtypeStruct((NUM_TOKENS, K), jnp.float32),
            jax.ShapeDtypeStruct((NUM_TOKENS, K), jnp.int32),
        ),
        mesh=mesh,
        compiler_params=pltpu.CompilerParams(needs_layout_passes=False),
        scratch_types=[
            pltpu.VMEM((TPW, NUM_EXPERTS), jnp.float32),
            pltpu.VMEM((NUM_EXPERTS,), jnp.float32),
            pltpu.VMEM((TPW // 4, K), jnp.float32),
            pltpu.VMEM((TPW // 4, K), jnp.int32),
            pltpu.VMEM((NUM_EXPERTS * SSTR,), jnp.float32),
            pltpu.VMEM((NCAND * L,), jnp.float32),
            pltpu.VMEM((TOPK_GROUPS * L,), jnp.int32),
        ],
    )(_tec_kernel)
    return run(router_logits, correction_bias)


# speedup vs baseline: 1.4317x; 1.1524x over previous
import functools

import jax
import jax.numpy as jnp
from jax import lax
from jax.experimental import pallas as pl
from jax.experimental.pallas import tpu as pltpu
from jax.experimental.pallas import tpu_sc as plsc

NUM_TOKENS = 16384
NUM_EXPERTS = 64
NUM_GROUPS = 8
GROUP_SIZE = NUM_EXPERTS // NUM_GROUPS
TOPK_GROUPS = 4
NCAND = TOPK_GROUPS * GROUP_SIZE
K = 8
SCALE = 2.5

NC = 2
NS = 16
L = 16
NW = NC * NS
TPW = NUM_TOKENS // NW
TILE = L
NT = TPW // TILE
NE = 8
ETILES = NT // NE
ESZ = TPW // NE
SSTR = L + 1


def _tec_kernel(logits_hbm, bias_hbm, w_hbm, id_hbm,
                raw_v, bias_v, sbuf_v, cbuf_v, gmap_v,
                wout0, iout0, wout1, iout1,
                insem, wsem0, isem0, wsem1, isem1):
    wid = lax.axis_index("s") * NC + lax.axis_index("c")
    base = wid * TPW

    half = TPW // 2
    pltpu.sync_copy(logits_hbm.at[pl.ds(base, half)], raw_v.at[pl.ds(0, half)])
    pltpu.async_copy(logits_hbm.at[pl.ds(base + half, half)],
                     raw_v.at[pl.ds(half, half)], insem)
    pltpu.sync_copy(bias_hbm, bias_v)

    iota = lax.iota(jnp.int32, L)
    neg_inf = jnp.full((L,), -jnp.inf, jnp.float32)
    bias_chunks = [bias_v[pl.ds(c * L, L)] for c in range(NUM_EXPERTS // L)]

    def tile_body(t, carry):
        @pl.when(t == NT // 2)
        def _wait_in():
            pltpu.make_async_copy(
                logits_hbm.at[pl.ds(base + half, half)],
                raw_v.at[pl.ds(half, half)], insem).wait()

        for buf, wv, iv, wsm, ism in ((0, wout0, iout0, wsem0, isem0),
                                      (1, wout1, iout1, wsem1, isem1)):
            @pl.when(jnp.logical_and(
                jnp.logical_and(t % ETILES == 0, t // ETILES >= 2),
                (t // ETILES) % 2 == buf))
            def _reclaim(wv=wv, iv=iv, wsm=wsm, ism=ism):
                pltpu.make_async_copy(wv, w_hbm.at[pl.ds(base, ESZ)],
                                      wsm).wait()
                pltpu.make_async_copy(iv, id_hbm.at[pl.ds(base, ESZ)],
                                      ism).wait()

        rows = t * TILE + iota
        coliota = iota

        for c in range(TILE * NUM_EXPERTS // L):
            tok = jnp.full((L,), c // 4, jnp.int32) + t * TILE
            v = plsc.load_gather(raw_v, [tok, (c % 4) * L + coliota])
            v = v + bias_chunks[c % 4]
            dst = ((c % 4) * L + iota) * SSTR + (c // 4)
            plsc.store_scatter(sbuf_v, [dst], v)

        m1 = [neg_inf] * NUM_GROUPS
        m2 = [neg_inf] * NUM_GROUPS
        srow = []
        for e in range(NUM_EXPERTS):
            s = plsc.load_gather(sbuf_v, [iota + e * SSTR])
            srow.append(s)
            g = e // GROUP_SIZE
            m2[g] = jnp.maximum(m2[g], jnp.minimum(m1[g], s))
            m1[g] = jnp.maximum(m1[g], s)
        gs = [m1[g] + m2[g] for g in range(NUM_GROUPS)]

        sel = []
        for g in range(NUM_GROUPS):
            cnt = jnp.zeros((L,), jnp.int32)
            for h in range(NUM_GROUPS):
                if h == g:
                    continue
                beats = gs[h] > gs[g]
                if h < g:
                    beats = jnp.logical_or(beats, gs[h] == gs[g])
                cnt = cnt + jnp.where(beats, 1, 0)
            sel.append(cnt < TOPK_GROUPS)

        rank = jnp.zeros((L,), jnp.int32)
        gbase = []
        for g in range(NUM_GROUPS):
            gbase.append(rank * (GROUP_SIZE * L) + iota)
            plsc.store_scatter(gmap_v, [rank * L + iota],
                               jnp.full((L,), g, jnp.int32), mask=sel[g])
            rank = rank + jnp.where(sel[g], 1, 0)
        for e in range(NUM_EXPERTS):
            g = e // GROUP_SIZE
            plsc.store_scatter(cbuf_v, [gbase[g] + (e % GROUP_SIZE) * L],
                               srow[e], mask=sel[g])

        cand = [cbuf_v[pl.ds(i * L, L)] for i in range(NCAND)]
        ws = []
        bis = []
        for k in range(K):
            vals = list(cand)
            idxs = [jnp.full((L,), i, jnp.int32) for i in range(NCAND)]
            n = NCAND
            while n > 1:
                nv, ni = [], []
                for i in range(0, n, 2):
                    better = vals[i + 1] > vals[i]
                    nv.append(jnp.where(better, vals[i + 1], vals[i]))
                    ni.append(jnp.where(better, idxs[i + 1], idxs[i]))
                vals, idxs, n = nv, ni, n // 2
            bslot = idxs[0]
            for i in range(NCAND):
                cand[i] = jnp.where(bslot == i, neg_inf, cand[i])
            gm = plsc.load_gather(gmap_v, [(bslot // GROUP_SIZE) * L + iota])
            bi = gm * GROUP_SIZE + (bslot % GROUP_SIZE)
            ws.append(plsc.load_gather(raw_v, [rows, bi]))
            bis.append(bi)

        lvl = list(ws)
        while len(lvl) > 1:
            h = len(lvl) // 2
            lvl = [lvl[i] + lvl[i + h] for i in range(h)]
        wsum = lvl[0]
        inv = SCALE / wsum
        q = t // ETILES
        par = q % 2
        rlocal = (t % ETILES) * TILE + iota
        for buf, wv, iv, wsm, ism in ((0, wout0, iout0, wsem0, isem0),
                                      (1, wout1, iout1, wsem1, isem1)):
            @pl.when(par == buf)
            def _stage(wv=wv, iv=iv):
                for k in range(K):
                    kcol = jnp.full((L,), k, jnp.int32)
                    plsc.store_scatter(wv, [rlocal, kcol], ws[k] * inv)
                    plsc.store_scatter(iv, [rlocal, kcol], bis[k])

            @pl.when(jnp.logical_and(t % ETILES == ETILES - 1, par == buf))
            def _flush(wv=wv, iv=iv, wsm=wsm, ism=ism):
                qbase = base + q * ESZ
                pltpu.async_copy(wv, w_hbm.at[pl.ds(qbase, ESZ)], wsm)
                pltpu.async_copy(iv, id_hbm.at[pl.ds(qbase, ESZ)], ism)
        return carry

    lax.fori_loop(0, NT, tile_body, 0)

    pltpu.make_async_copy(wout0, w_hbm.at[pl.ds(base, ESZ)], wsem0).wait()
    pltpu.make_async_copy(iout0, id_hbm.at[pl.ds(base, ESZ)], isem0).wait()
    pltpu.make_async_copy(wout1, w_hbm.at[pl.ds(base, ESZ)], wsem1).wait()
    pltpu.make_async_copy(iout1, id_hbm.at[pl.ds(base, ESZ)], isem1).wait()


@jax.jit
def kernel(router_logits, correction_bias):
    mesh = plsc.VectorSubcoreMesh(core_axis_name="c", subcore_axis_name="s")
    run = functools.partial(
        pl.kernel,
        out_type=(
            jax.ShapeDtypeStruct((NUM_TOKENS, K), jnp.float32),
            jax.ShapeDtypeStruct((NUM_TOKENS, K), jnp.int32),
        ),
        mesh=mesh,
        compiler_params=pltpu.CompilerParams(needs_layout_passes=False),
        scratch_types=[
            pltpu.VMEM((TPW, NUM_EXPERTS), jnp.float32),
            pltpu.VMEM((NUM_EXPERTS,), jnp.float32),
            pltpu.VMEM((NUM_EXPERTS * SSTR,), jnp.float32),
            pltpu.VMEM((NCAND * L,), jnp.float32),
            pltpu.VMEM((TOPK_GROUPS * L,), jnp.int32),
            pltpu.VMEM((ESZ, K), jnp.float32),
            pltpu.VMEM((ESZ, K), jnp.int32),
            pltpu.VMEM((ESZ, K), jnp.float32),
            pltpu.VMEM((ESZ, K), jnp.int32),
            pltpu.SemaphoreType.DMA,
            pltpu.SemaphoreType.DMA,
            pltpu.SemaphoreType.DMA,
            pltpu.SemaphoreType.DMA,
            pltpu.SemaphoreType.DMA,
        ],
    )(_tec_kernel)
    return run(router_logits, correction_bias)
